# baseline (speedup 1.0000x reference)
import functools

import jax
import jax.numpy as jnp
import numpy as np
from jax import lax
from jax.experimental import pallas as pl
from jax.experimental.pallas import tpu as pltpu
from jax.experimental.pallas import tpu_sc as plsc

_NUM_LEVELS = 16
_LEVEL_DIM = 2
_N = 262144
_NW = 32
_PPW = _N // _NW
_CHUNK = 256
_NCHUNK = _PPW // _CHUNK
_VPC = _CHUNK // 16

_P1 = int(np.int32(np.uint32(2654435761)))
_P2 = int(np.int32(np.uint32(805459861)))
_HASH_MASK = (1 << 19) - 1

_DENSE_OFFS = (0, 4913, 40850)
_HASH_OFF0 = 315475
_HASH_SIZE = 1 << 19
_BOUNCE_W = 8192
_DENSE_WORDS = 16 * 5 * _BOUNCE_W


def _pos_frac(coords_v, sl, scale):
    g = []
    f = []
    for d in range(3):
        v = coords_v[d][sl]
        x = (v + 1.0) * 0.5
        pos = x * scale + 0.5
        gi = pos.astype(jnp.int32)
        g.append(gi)
        f.append(pos - gi.astype(jnp.float32))
    return g, f


def _phase1(ci, scale, dense_strides, level_off, hbm_woff, bufs, coords_v):
    cb = ci * _CHUNK
    idxa = bufs["idxa"]
    idxb = bufs["idxb"]
    wbuf = bufs["w"]

    def vec_body(j, carry):
        sl = pl.ds(cb + 16 * j, 16)
        g, f = _pos_frac(coords_v, sl, scale)
        if dense_strides is None:
            tx = (g[0], g[0] + 1)
            ty0 = g[1] * _P1
            ty = (ty0, ty0 + _P1)
            tz0 = g[2] * _P2
            tz = (tz0, tz0 + _P2)
        else:
            stride, s2 = dense_strides
            tx = (g[0], g[0] + 1)
            ty0 = g[1] * stride
            ty = (ty0, ty0 + stride)
            tz0 = g[2] * s2
            tz = (tz0, tz0 + s2)
        wx = (1.0 - f[0], f[0])
        wy = (1.0 - f[1], f[1])
        wz = (1.0 - f[2], f[2])
        for c in range(8):
            b0, b1, b2 = c & 1, (c >> 1) & 1, (c >> 2) & 1
            if dense_strides is None:
                idx = ((tx[b0] ^ ty[b1] ^ tz[b2]) & _HASH_MASK) + level_off
            else:
                idx = tx[b0] + ty[b1] + tz[b2] + level_off
            widx = idx * 2
            w = wx[b0] * wy[b1] * wz[b2]
            if c < 4:
                idxa[pl.ds(c * _CHUNK + 16 * j, 16)] = widx
                idxa[pl.ds((4 + c) * _CHUNK + 16 * j, 16)] = widx + 1
            else:
                gw = widx + hbm_woff
                idxb[pl.ds((c - 4) * _CHUNK + 16 * j, 16)] = gw
                idxb[pl.ds(c * _CHUNK + 16 * j, 16)] = gw + 1
            wbuf[pl.ds(c * _CHUNK + 16 * j, 16)] = w
        return carry

    lax.fori_loop(0, _VPC, vec_body, 0)


def _fire(bufs, shared, emb_all_hbm):
    pltpu.async_copy(shared.at[bufs["idxa"]], bufs["rowsa"], bufs["gsema"])
    pltpu.async_copy(emb_all_hbm.at[bufs["idxb"]], bufs["rowsb"],
                     bufs["gsemb"])


def _drain_gather(bufs, dummy_hbm):
    dummy = dummy_hbm.at[pl.ds(0, 8 * _CHUNK)]
    pltpu.make_async_copy(dummy, bufs["rowsa"], bufs["gsema"]).wait()
    pltpu.make_async_copy(dummy, bufs["rowsb"], bufs["gsemb"]).wait()


def _phase3_store(ci, out_base, bufs, out_hbm, osem):
    rowsa = bufs["rowsa"]
    rowsb = bufs["rowsb"]
    wbuf = bufs["w"]

    def vec_body(j, carry):
        jsl = pl.ds(16 * j, 16)
        acc0 = jnp.zeros((16,), jnp.float32)
        acc1 = jnp.zeros((16,), jnp.float32)
        for c in range(8):
            rows = rowsa if c < 4 else rowsb
            cc = c if c < 4 else c - 4
            csl = pl.ds(cc * _CHUNK + 16 * j, 16)
            osl = pl.ds((4 + cc) * _CHUNK + 16 * j, 16)
            w = wbuf[pl.ds(c * _CHUNK + 16 * j, 16)]
            acc0 = acc0 + w * rows[csl]
            acc1 = acc1 + w * rows[osl]
        bufs["st0"][jsl] = acc0
        bufs["st1"][jsl] = acc1
        return carry

    lax.fori_loop(0, _VPC, vec_body, 0)
    ob = out_base + ci * _CHUNK
    pltpu.async_copy(bufs["st0"], out_hbm.at[pl.ds(ob, _CHUNK)], osem)
    pltpu.async_copy(bufs["st1"], out_hbm.at[pl.ds(ob + _N, _CHUNK)], osem)


def _drain_out(bufs, out_hbm, osem):
    dummy = out_hbm.at[pl.ds(0, _CHUNK)]
    pltpu.make_async_copy(dummy, bufs["st0"], osem).wait()
    pltpu.make_async_copy(dummy, bufs["st1"], osem).wait()


@functools.partial(
    pl.kernel,
    out_type=jax.ShapeDtypeStruct((_NUM_LEVELS * _LEVEL_DIM * _N,),
                                  jnp.float32),
    mesh=plsc.VectorSubcoreMesh(core_axis_name="c", subcore_axis_name="s"),
    scratch_types=(
        [pltpu.VMEM((_PPW,), jnp.float32)] * 3
        + [pltpu.VMEM((8 * _CHUNK,), jnp.int32)] * 4
        + [pltpu.VMEM((8 * _CHUNK,), jnp.float32)] * 2
        + [pltpu.VMEM((8 * _CHUNK,), jnp.float32)] * 4
        + [pltpu.VMEM((_CHUNK,), jnp.float32)] * 4
        + [pltpu.SemaphoreType.DMA] * 6
        + [pltpu.VMEM_SHARED((_HASH_SIZE * 2,), jnp.float32)]
        + [pltpu.VMEM((_BOUNCE_W,), jnp.float32)]
    ),
)
def _hash_grid(xs_hbm, ys_hbm, zs_hbm, emb_dense_hbm, emb_hash_hbm,
               emb_all_hbm, out_hbm, *scratch):
    coords_v = scratch[0:3]
    slots = []
    for s in range(2):
        slots.append({
            "idxa": scratch[3 + s],
            "idxb": scratch[5 + s],
            "w": scratch[7 + s],
            "rowsa": scratch[9 + s],
            "rowsb": scratch[11 + s],
            "st0": scratch[13 + 2 * s],
            "st1": scratch[14 + 2 * s],
            "gsema": scratch[17 + s],
            "gsemb": scratch[19 + s],
            "osem": scratch[21 + s],
        })
    shared = scratch[23]
    bounce_v = scratch[24]

    wid = lax.axis_index("s") * 2 + lax.axis_index("c")
    base = wid * _PPW
    sid = lax.axis_index("s")
    for d, src in enumerate((xs_hbm, ys_hbm, zs_hbm)):
        pltpu.sync_copy(src.at[pl.ds(base, _PPW)], coords_v[d])

    def stage_table(src_hbm, word_off, nblk):
        soff = sid * (nblk * _BOUNCE_W)
        plsc.subcore_barrier()

        def blk_body(t, carry):
            o = soff + t * _BOUNCE_W
            pltpu.sync_copy(src_hbm.at[pl.ds(word_off + o, _BOUNCE_W)],
                            bounce_v)
            pltpu.sync_copy(bounce_v, shared.at[pl.ds(o, _BOUNCE_W)])
            return carry

        lax.fori_loop(0, nblk, blk_body, 0)
        plsc.subcore_barrier()

    def run_level(lvl, scale, dense_strides, off, hbm_woff):
        out_base = lvl * (2 * _N) + base
        b0, b1 = slots[0], slots[1]
        _phase1(0, scale, dense_strides, off, hbm_woff, b0, coords_v)
        _fire(b0, shared, emb_all_hbm)

        def k_iter(k, carry):
            _phase1(2 * k + 1, scale, dense_strides, off, hbm_woff, b1,
                    coords_v)
            _fire(b1, shared, emb_all_hbm)
            _drain_gather(b0, out_hbm)

            @pl.when(k >= 1)
            def _():
                _drain_out(b0, out_hbm, b0["osem"])
            _phase3_store(2 * k, out_base, b0, out_hbm, b0["osem"])

            @pl.when(k < _NCHUNK // 2 - 1)
            def _():
                _phase1(2 * k + 2, scale, dense_strides, off, hbm_woff, b0,
                        coords_v)
                _fire(b0, shared, emb_all_hbm)
            _drain_gather(b1, out_hbm)

            @pl.when(k >= 1)
            def _():
                _drain_out(b1, out_hbm, b1["osem"])
            _phase3_store(2 * k + 1, out_base, b1, out_hbm, b1["osem"])
            return carry

        lax.fori_loop(0, _NCHUNK // 2, k_iter, 0)
        _drain_out(b0, out_hbm, b0["osem"])
        _drain_out(b1, out_hbm, b1["osem"])

    stage_table(emb_dense_hbm, 0, _DENSE_WORDS // (16 * _BOUNCE_W))

    def dense_level(lvl, carry):
        stride = lax.shift_left(16, lvl) + 1
        scale = (stride - 2).astype(jnp.float32)
        off = jnp.where(lvl == 0, 0,
                        jnp.where(lvl == 1, _DENSE_OFFS[1], _DENSE_OFFS[2]))
        run_level(lvl, scale, (stride, stride * stride), off, 0)
        return carry

    lax.fori_loop(0, 3, dense_level, 0)

    def hash_level(lvl, carry):
        scale = (lax.shift_left(16, lvl) - 1).astype(jnp.float32)
        stage_table(emb_hash_hbm, (lvl - 3) * (2 * _HASH_SIZE),
                    2 * _HASH_SIZE // (16 * _BOUNCE_W))
        run_level(lvl, scale, None, 0,
                  2 * (_HASH_OFF0 + (lvl - 3) * _HASH_SIZE))
        return carry

    lax.fori_loop(3, _NUM_LEVELS, hash_level, 0)


def kernel(inputs, embeddings):
    xs = inputs[:, 0]
    ys = inputs[:, 1]
    zs = inputs[:, 2]
    emb_flat = embeddings.reshape(-1)
    emb_dense = emb_flat[:_DENSE_WORDS]
    emb_hash = emb_flat[2 * _HASH_OFF0:]
    out = _hash_grid(xs, ys, zs, emb_dense, emb_hash, emb_flat)
    out = out.reshape(_NUM_LEVELS, _LEVEL_DIM, _N)
    return out.transpose(2, 0, 1).reshape(_N, _NUM_LEVELS * _LEVEL_DIM)


# speedup vs baseline: 1.1254x; 1.1254x over previous
import functools

import jax
import jax.numpy as jnp
import numpy as np
from jax import lax
from jax.experimental import pallas as pl
from jax.experimental.pallas import tpu as pltpu
from jax.experimental.pallas import tpu_sc as plsc

_NUM_LEVELS = 16
_LEVEL_DIM = 2
_N = 262144
_NW = 32
_PPW = _N // _NW
_CHUNK = 256
_NCHUNK = _PPW // _CHUNK
_VPC = _CHUNK // 16

_P1 = int(np.int32(np.uint32(2654435761)))
_P2 = int(np.int32(np.uint32(805459861)))
_HASH_MASK = (1 << 19) - 1

_DENSE_OFFS = (0, 4913, 40850)
_HASH_OFF0 = 315475
_HASH_SIZE = 1 << 19
_BOUNCE_W = 8192
_DENSE_WORDS = 16 * 5 * _BOUNCE_W


def _pos_frac(coords_v, sl, scale):
    g = []
    f = []
    for d in range(3):
        v = coords_v[d][sl]
        x = (v + 1.0) * 0.5
        pos = x * scale + 0.5
        gi = pos.astype(jnp.int32)
        g.append(gi)
        f.append(pos - gi.astype(jnp.float32))
    return g, f


def _phase1(ci, scale, dense_strides, level_off, bufs, coords_v):
    cb = ci * _CHUNK
    idx2 = bufs["idx"]
    wbuf = bufs["w"]

    def vec_body(j, carry):
        sl = pl.ds(cb + 16 * j, 16)
        g, f = _pos_frac(coords_v, sl, scale)
        if dense_strides is None:
            tx = (g[0], g[0] + 1)
            ty0 = g[1] * _P1
            ty = (ty0, ty0 + _P1)
            tz0 = g[2] * _P2
            tz = (tz0, tz0 + _P2)
        else:
            stride, s2 = dense_strides
            tx = (g[0], g[0] + 1)
            ty0 = g[1] * stride
            ty = (ty0, ty0 + stride)
            tz0 = g[2] * s2
            tz = (tz0, tz0 + s2)
        wx = (1.0 - f[0], f[0])
        wy = (1.0 - f[1], f[1])
        wz = (1.0 - f[2], f[2])
        for c in range(8):
            b0, b1, b2 = c & 1, (c >> 1) & 1, (c >> 2) & 1
            if dense_strides is None:
                idx = ((tx[b0] ^ ty[b1] ^ tz[b2]) & _HASH_MASK) + level_off
            else:
                idx = tx[b0] + ty[b1] + tz[b2] + level_off
            widx = idx * 2
            w = wx[b0] * wy[b1] * wz[b2]
            idx2[pl.ds(c * _CHUNK + 16 * j, 16)] = widx
            idx2[pl.ds((8 + c) * _CHUNK + 16 * j, 16)] = widx + 1
            wbuf[pl.ds(c * _CHUNK + 16 * j, 16)] = w
        return carry

    lax.fori_loop(0, _VPC, vec_body, 0)


def _fire(bufs, shared, sem):
    pltpu.async_copy(shared.at[bufs["idx"]], bufs["rows"], sem)


def _drain_gather(bufs, dummy_hbm, sem):
    pltpu.make_async_copy(dummy_hbm.at[pl.ds(0, 16 * _CHUNK)],
                          bufs["rows"], sem).wait()


def _phase3_store(ci, out_base, bufs, out_hbm, osem):
    rows = bufs["rows"]
    wbuf = bufs["w"]

    def vec_body(j, carry):
        jsl = pl.ds(16 * j, 16)
        acc0 = jnp.zeros((16,), jnp.float32)
        acc1 = jnp.zeros((16,), jnp.float32)
        for c in range(8):
            csl = pl.ds(c * _CHUNK + 16 * j, 16)
            osl = pl.ds((8 + c) * _CHUNK + 16 * j, 16)
            w = wbuf[csl]
            acc0 = acc0 + w * rows[csl]
            acc1 = acc1 + w * rows[osl]
        bufs["st0"][jsl] = acc0
        bufs["st1"][jsl] = acc1
        return carry

    lax.fori_loop(0, _VPC, vec_body, 0)
    ob = out_base + ci * _CHUNK
    pltpu.async_copy(bufs["st0"], out_hbm.at[pl.ds(ob, _CHUNK)], osem)
    pltpu.async_copy(bufs["st1"], out_hbm.at[pl.ds(ob + _N, _CHUNK)], osem)


def _drain_out(bufs, out_hbm, osem):
    dummy = out_hbm.at[pl.ds(0, _CHUNK)]
    pltpu.make_async_copy(dummy, bufs["st0"], osem).wait()
    pltpu.make_async_copy(dummy, bufs["st1"], osem).wait()


@functools.partial(
    pl.kernel,
    out_type=jax.ShapeDtypeStruct((_NUM_LEVELS * _LEVEL_DIM * _N,),
                                  jnp.float32),
    mesh=plsc.VectorSubcoreMesh(core_axis_name="c", subcore_axis_name="s"),
    scratch_types=(
        [pltpu.VMEM((_PPW,), jnp.float32)] * 3
        + [pltpu.VMEM((16 * _CHUNK,), jnp.int32)] * 2
        + [pltpu.VMEM((8 * _CHUNK,), jnp.float32)] * 2
        + [pltpu.VMEM((16 * _CHUNK,), jnp.float32)] * 2
        + [pltpu.VMEM((_CHUNK,), jnp.float32)] * 4
        + [pltpu.SemaphoreType.DMA] * 4
        + [pltpu.VMEM_SHARED((_HASH_SIZE * 2,), jnp.float32)]
        + [pltpu.VMEM((_BOUNCE_W,), jnp.float32)]
    ),
)
def _hash_grid(xs_hbm, ys_hbm, zs_hbm, emb_dense_hbm, emb_hash_hbm, out_hbm,
               *scratch):
    coords_v = scratch[0:3]
    slots = []
    for s in range(2):
        slots.append({
            "idx": scratch[3 + s],
            "w": scratch[5 + s],
            "rows": scratch[7 + s],
            "st0": scratch[9 + 2 * s],
            "st1": scratch[10 + 2 * s],
            "gsem": scratch[13 + s],
            "osem": scratch[15 + s],
        })
    shared = scratch[17]
    bounce_v = scratch[18]

    wid = lax.axis_index("s") * 2 + lax.axis_index("c")
    base = wid * _PPW
    sid = lax.axis_index("s")
    for d, src in enumerate((xs_hbm, ys_hbm, zs_hbm)):
        pltpu.sync_copy(src.at[pl.ds(base, _PPW)], coords_v[d])

    def stage_table(src_hbm, word_off, nblk):
        soff = sid * (nblk * _BOUNCE_W)
        plsc.subcore_barrier()

        def blk_body(t, carry):
            o = soff + t * _BOUNCE_W
            pltpu.sync_copy(src_hbm.at[pl.ds(word_off + o, _BOUNCE_W)],
                            bounce_v)
            pltpu.sync_copy(bounce_v, shared.at[pl.ds(o, _BOUNCE_W)])
            return carry

        lax.fori_loop(0, nblk, blk_body, 0)
        plsc.subcore_barrier()

    def run_level(lvl, scale, dense_strides, off):
        out_base = lvl * (2 * _N) + base
        b0, b1 = slots[0], slots[1]
        _phase1(0, scale, dense_strides, off, b0, coords_v)
        _fire(b0, shared, b0["gsem"])

        def k_iter(k, carry):
            _phase1(2 * k + 1, scale, dense_strides, off, b1, coords_v)
            _fire(b1, shared, b1["gsem"])
            _drain_gather(b0, out_hbm, b0["gsem"])

            @pl.when(k >= 1)
            def _():
                _drain_out(b0, out_hbm, b0["osem"])
            _phase3_store(2 * k, out_base, b0, out_hbm, b0["osem"])

            @pl.when(k < _NCHUNK // 2 - 1)
            def _():
                _phase1(2 * k + 2, scale, dense_strides, off, b0, coords_v)
                _fire(b0, shared, b0["gsem"])
            _drain_gather(b1, out_hbm, b1["gsem"])

            @pl.when(k >= 1)
            def _():
                _drain_out(b1, out_hbm, b1["osem"])
            _phase3_store(2 * k + 1, out_base, b1, out_hbm, b1["osem"])
            return carry

        lax.fori_loop(0, _NCHUNK // 2, k_iter, 0)
        _drain_out(b0, out_hbm, b0["osem"])
        _drain_out(b1, out_hbm, b1["osem"])

    stage_table(emb_dense_hbm, 0, _DENSE_WORDS // (16 * _BOUNCE_W))

    def dense_level(lvl, carry):
        stride = lax.shift_left(16, lvl) + 1
        scale = (stride - 2).astype(jnp.float32)
        off = jnp.where(lvl == 0, 0,
                        jnp.where(lvl == 1, _DENSE_OFFS[1], _DENSE_OFFS[2]))
        run_level(lvl, scale, (stride, stride * stride), off)
        return carry

    lax.fori_loop(0, 3, dense_level, 0)

    def hash_level(lvl, carry):
        scale = (lax.shift_left(16, lvl) - 1).astype(jnp.float32)
        stage_table(emb_hash_hbm, (lvl - 3) * (2 * _HASH_SIZE),
                    2 * _HASH_SIZE // (16 * _BOUNCE_W))
        run_level(lvl, scale, None, 0)
        return carry

    lax.fori_loop(3, _NUM_LEVELS, hash_level, 0)


def kernel(inputs, embeddings):
    xs = inputs[:, 0]
    ys = inputs[:, 1]
    zs = inputs[:, 2]
    emb_flat = embeddings.reshape(-1)
    emb_dense = emb_flat[:_DENSE_WORDS]
    emb_hash = emb_flat[2 * _HASH_OFF0:]
    out = _hash_grid(xs, ys, zs, emb_dense, emb_hash)
    out = out.reshape(_NUM_LEVELS, _LEVEL_DIM, _N)
    return out.transpose(2, 0, 1).reshape(_N, _NUM_LEVELS * _LEVEL_DIM)
